# baseline (speedup 1.0000x reference)
import functools

import jax
import jax.numpy as jnp
from jax import lax
from jax.experimental import pallas as pl
from jax.experimental.pallas import tpu as pltpu
from jax.experimental.pallas import tpu_sc as plsc

NC, NS, L = 2, 16, 16
NW = NC * NS
N = 10000
D = 128
DH = D // NC
G = 128
GH = G // NC
PN = 10240
CHUNK = 128
RPT = PN // NS

_MESH = plsc.VectorSubcoreMesh(
    core_axis_name="c", subcore_axis_name="s", num_cores=NC, num_subcores=NS)


def _pad_edges(idx, total, fill):
    return jnp.concatenate(
        [idx, jnp.full((total - idx.shape[0],), fill, dtype=idx.dtype)])


def _k1_body(dste_hbm, batp_hbm, zeros_hbm, degp_hbm, cntp_hbm,
             dst_v, bat_v, ones_v, deg_sh, cnt_sh, sem):
    cid = lax.axis_index("c")
    sid = lax.axis_index("s")
    wid = cid * NS + sid
    nde = dst_v.shape[0]
    nbt = bat_v.shape[0]

    pltpu.sync_copy(dste_hbm.at[wid], dst_v)
    pltpu.sync_copy(batp_hbm.at[wid], bat_v)
    for l in range(CHUNK // L):
        ones_v[pl.ds(l * L, L)] = jnp.ones((L,), jnp.float32)
    pltpu.sync_copy(zeros_hbm.at[pl.ds(0, PN // NS)],
                    deg_sh.at[pl.ds(sid * (PN // NS), PN // NS)])

    @pl.when(sid == 0)
    def _():
        pltpu.sync_copy(zeros_hbm.at[pl.ds(0, 256)], cnt_sh)

    plsc.subcore_barrier()

    def deg_step(j, _):
        pltpu.sync_copy(ones_v, deg_sh.at[dst_v.at[j]], add=True)
        return 0

    lax.fori_loop(0, nde, deg_step, 0)

    def cnt_step(j, _):
        pltpu.sync_copy(ones_v, cnt_sh.at[bat_v.at[j]], add=True)
        return 0

    lax.fori_loop(0, nbt, cnt_step, 0)
    plsc.subcore_barrier()

    pltpu.sync_copy(deg_sh.at[pl.ds(sid * (PN // NS), PN // NS)],
                    degp_hbm.at[cid, pl.ds(sid * (PN // NS), PN // NS)])

    @pl.when(sid == 0)
    def _():
        pltpu.sync_copy(cnt_sh, cntp_hbm.at[cid])


def _k1_call(dste, batp, zeros1d, n_dchunks, n_bchunks):
    return pl.kernel(
        _k1_body,
        out_type=[jax.ShapeDtypeStruct((NC, PN), jnp.float32),
                  jax.ShapeDtypeStruct((NC, 256), jnp.float32)],
        mesh=_MESH,
        scratch_types=[
            pltpu.VMEM((n_dchunks, CHUNK), jnp.int32),
            pltpu.VMEM((n_bchunks, CHUNK), jnp.int32),
            pltpu.VMEM((CHUNK,), jnp.float32),
            pltpu.VMEM_SHARED((PN,), jnp.float32),
            pltpu.VMEM_SHARED((256,), jnp.float32),
            pltpu.SemaphoreType.DMA,
        ],
    )(dste, batp, zeros1d)


def _k4_body(src_hbm, dst_hbm, ycat_hbm, zeros2d_hbm, accp_hbm,
             src_v, dst_v, rows_v, acc_sh, gsem, ssem):
    cid = lax.axis_index("c")
    sid = lax.axis_index("s")
    nch = src_v.shape[1]
    off = cid * PN

    pltpu.sync_copy(src_hbm.at[pl.ds(sid * 2, 2)], src_v)
    pltpu.sync_copy(dst_hbm.at[pl.ds(sid * 2, 2)], dst_v)
    pltpu.sync_copy(zeros2d_hbm, acc_sh.at[pl.ds(sid * RPT, RPT)])

    def bias_step(k, _):
        w = k // nch
        jj = k - w * nch
        for l in range(CHUNK // L):
            src_v[w, jj, pl.ds(l * L, L)] = (
                src_v[w, jj, pl.ds(l * L, L)] + off)
        return 0

    lax.fori_loop(0, 2 * nch, bias_step, 0)
    plsc.subcore_barrier()

    nt = 2 * nch

    def _gather(k, buf):
        w = k // nch
        jj = k - w * nch
        pltpu.async_copy(ycat_hbm.at[src_v.at[w, jj]], rows_v.at[buf], gsem)

    for b in range(3):
        _gather(b, b)

    def step(k, _):
        w = k // nch
        jj = k - w * nch
        cur = lax.rem(k, 4)
        pltpu.make_async_copy(ycat_hbm.at[src_v.at[w, jj]], rows_v.at[cur],
                              gsem).wait()
        pltpu.async_copy(rows_v.at[cur], acc_sh.at[dst_v.at[w, jj]], ssem,
                         add=True)

        @pl.when(k >= 1)
        def _():
            km = k - 1
            wm = km // nch
            jm = km - wm * nch
            pltpu.make_async_copy(rows_v.at[lax.rem(km, 4)],
                                  acc_sh.at[dst_v.at[wm, jm]], ssem).wait()

        @pl.when(k + 3 < nt)
        def _():
            k3 = k + 3
            w3 = k3 // nch
            j3 = k3 - w3 * nch
            pltpu.async_copy(ycat_hbm.at[src_v.at[w3, j3]],
                             rows_v.at[lax.rem(k3, 4)], gsem)

        return 0

    lax.fori_loop(0, nt, step, 0)
    pltpu.make_async_copy(rows_v.at[lax.rem(nt - 1, 4)],
                          acc_sh.at[dst_v.at[1, nch - 1]], ssem).wait()
    plsc.subcore_barrier()

    pltpu.sync_copy(acc_sh.at[pl.ds(sid * RPT, RPT)],
                    accp_hbm.at[cid, pl.ds(sid * RPT, RPT)])


def _k4_call(src1, dst1, ycat, zeros2d, n_chunks):
    return pl.kernel(
        _k4_body,
        out_type=jax.ShapeDtypeStruct((NC, PN, DH), jnp.float32),
        mesh=_MESH,
        scratch_types=[
            pltpu.VMEM((2, n_chunks, CHUNK), jnp.int32),
            pltpu.VMEM((2, n_chunks, CHUNK), jnp.int32),
            pltpu.VMEM((4, CHUNK, DH), jnp.float32),
            pltpu.VMEM_SHARED((PN, DH), jnp.float32),
            pltpu.SemaphoreType.DMA,
            pltpu.SemaphoreType.DMA,
        ],
        compiler_params=pltpu.CompilerParams(use_tc_tiling_on_sc=False),
    )(src1, dst1, ycat, zeros2d)


def _k3_body(src_hbm, dst_hbm, dinv_hbm, r_hbm, ibase_hbm, zeros_hbm, cp_hbm,
             src_v, dst_v, dvs_v, dvd_v, ib_v, val_v, idx_v, c_sh, gsem, ssem):
    cid = lax.axis_index("c")
    sid = lax.axis_index("s")
    nch = src_v.shape[1]
    lo = cid * GH * PN

    pltpu.sync_copy(src_hbm.at[pl.ds(sid * 2, 2)], src_v)
    pltpu.sync_copy(dst_hbm.at[pl.ds(sid * 2, 2)], dst_v)
    csl = GH * PN // NS
    pltpu.sync_copy(zeros_hbm.at[pl.ds(0, csl)], c_sh.at[pl.ds(sid * csl, csl)])
    plsc.subcore_barrier()

    nt = 2 * nch

    def _gather(k, buf):
        w = k // nch
        jj = k - w * nch
        pltpu.async_copy(dinv_hbm.at[src_v.at[w, jj]], dvs_v.at[buf], gsem)
        pltpu.async_copy(r_hbm.at[dst_v.at[w, jj]], dvd_v.at[buf], gsem)
        pltpu.async_copy(ibase_hbm.at[dst_v.at[w, jj]], ib_v.at[buf], gsem)

    _gather(0, 0)
    _gather(1, 1)

    def _wait_gather(k, buf):
        w = k // nch
        jj = k - w * nch
        pltpu.make_async_copy(dinv_hbm.at[src_v.at[w, jj]], dvs_v.at[buf],
                              gsem).wait()
        pltpu.make_async_copy(r_hbm.at[dst_v.at[w, jj]], dvd_v.at[buf],
                              gsem).wait()
        pltpu.make_async_copy(ibase_hbm.at[dst_v.at[w, jj]], ib_v.at[buf],
                              gsem).wait()

    def step(j, _):
        w = j // nch
        jj = j - w * nch
        cur = lax.rem(j, 2)
        _wait_gather(j, cur)

        @pl.when(j >= 2)
        def _():
            pltpu.make_async_copy(val_v.at[cur], c_sh.at[idx_v.at[cur]],
                                  ssem).wait()

        for l in range(CHUNK // L):
            s = src_v[w, jj, pl.ds(l * L, L)]
            ib = ib_v[cur, pl.ds(l * L, L)]
            v = dvs_v[cur, pl.ds(l * L, L)] * dvd_v[cur, pl.ds(l * L, L)]
            keep = (ib >= lo) & (ib < lo + GH * PN)
            val_v[cur, pl.ds(l * L, L)] = jnp.where(keep, v, 0.0)
            idx_v[cur, pl.ds(l * L, L)] = jnp.where(keep, ib - lo + s, s)

        pltpu.async_copy(val_v.at[cur], c_sh.at[idx_v.at[cur]], ssem, add=True)

        @pl.when(j + 2 < nt)
        def _():
            _gather(j + 2, lax.rem(j + 2, 2))

        return 0

    lax.fori_loop(0, nt, step, 0)
    pltpu.make_async_copy(val_v.at[0], c_sh.at[idx_v.at[0]], ssem).wait()
    pltpu.make_async_copy(val_v.at[1], c_sh.at[idx_v.at[1]], ssem).wait()
    plsc.subcore_barrier()

    csl = GH * PN // NS
    pltpu.sync_copy(c_sh.at[pl.ds(sid * csl, csl)],
                    cp_hbm.at[cid, pl.ds(sid * csl, csl)])


def _k3_call(srcx, dstx, dinv, r, ibase, zeros1d, n_chunks):
    return pl.kernel(
        _k3_body,
        out_type=jax.ShapeDtypeStruct((NC, GH * PN), jnp.float32),
        mesh=_MESH,
        scratch_types=[
            pltpu.VMEM((2, n_chunks, CHUNK), jnp.int32),
            pltpu.VMEM((2, n_chunks, CHUNK), jnp.int32),
            pltpu.VMEM((2, CHUNK), jnp.float32),
            pltpu.VMEM((2, CHUNK), jnp.float32),
            pltpu.VMEM((2, CHUNK), jnp.int32),
            pltpu.VMEM((2, CHUNK), jnp.float32),
            pltpu.VMEM((2, CHUNK), jnp.int32),
            pltpu.VMEM_SHARED((GH * PN,), jnp.float32),
            pltpu.SemaphoreType.DMA,
            pltpu.SemaphoreType.DMA,
        ],
    )(srcx, dstx, dinv, r, ibase, zeros1d)


def _k2_body(x_ref, w1_ref, dinv_ref, bat_ref, icnt_ref, y_ref, r_ref):
    xw = jnp.dot(x_ref[...], w1_ref[...], preferred_element_type=jnp.float32)
    y = xw * dinv_ref[...]
    y_ref[0] = y[:, :DH]
    y_ref[1] = y[:, DH:]
    bm = bat_ref.shape[0]
    iota_g = lax.broadcasted_iota(jnp.int32, (bm, G), 1)
    oh = (bat_ref[...] == iota_g).astype(jnp.float32)
    icnt_node = jnp.dot(oh, icnt_ref[...], preferred_element_type=jnp.float32)
    r_ref[...] = dinv_ref[...] * icnt_node


def _k2_call(x_pad, w1, dinv_col, bat_col, icnt_col):
    bm = 512
    return pl.pallas_call(
        _k2_body,
        grid=(PN // bm,),
        in_specs=[
            pl.BlockSpec((bm, D), lambda i: (i, 0)),
            pl.BlockSpec((D, D), lambda i: (0, 0)),
            pl.BlockSpec((bm, 1), lambda i: (i, 0)),
            pl.BlockSpec((bm, 1), lambda i: (i, 0)),
            pl.BlockSpec((G, 1), lambda i: (0, 0)),
        ],
        out_specs=[pl.BlockSpec((2, bm, DH), lambda i: (0, i, 0)),
                   pl.BlockSpec((bm, 1), lambda i: (i, 0))],
        out_shape=[jax.ShapeDtypeStruct((2, PN, DH), jnp.float32),
                   jax.ShapeDtypeStruct((PN, 1), jnp.float32)],
    )(x_pad, w1, dinv_col, bat_col, icnt_col)


def _k5_body(accp_ref, y_ref, dinv_ref, b1_ref, h1_ref):
    acc = jnp.concatenate([accp_ref[0], accp_ref[1]], axis=1)
    y = jnp.concatenate([y_ref[0], y_ref[1]], axis=1)
    h1_ref[...] = jnp.maximum((acc + y) * dinv_ref[...] + b1_ref[...], 0.0)


def _k5_call(accp, ycat, dinv_col, b1_2d):
    bm = 1024
    return pl.pallas_call(
        _k5_body,
        grid=(PN // bm,),
        in_specs=[
            pl.BlockSpec((2, bm, DH), lambda i: (0, i, 0)),
            pl.BlockSpec((2, bm, DH), lambda i: (0, i, 0)),
            pl.BlockSpec((bm, 1), lambda i: (i, 0)),
            pl.BlockSpec((1, D), lambda i: (0, 0)),
        ],
        out_specs=pl.BlockSpec((bm, D), lambda i: (i, 0)),
        out_shape=jax.ShapeDtypeStruct((PN, D), jnp.float32),
    )(accp, ycat, dinv_col, b1_2d)


def _k6_body(c_ref, h1_ref, w2_ref, b2_ref, mask_ref, wr_ref, br_ref,
             out_ref, qacc_ref):
    i = pl.program_id(0)

    @pl.when(i == 0)
    def _():
        qacc_ref[...] = jnp.zeros_like(qacc_ref)

    qacc_ref[...] += jnp.dot(c_ref[...], h1_ref[...],
                             preferred_element_type=jnp.float32)

    @pl.when(i == pl.num_programs(0) - 1)
    def _():
        pooled = jnp.dot(qacc_ref[...], w2_ref[...],
                         preferred_element_type=jnp.float32) + b2_ref[...]
        pooled = pooled * mask_ref[...]
        h = jnp.maximum(pooled, 0.0)
        out_ref[...] = jnp.dot(h, wr_ref[...],
                               preferred_element_type=jnp.float32) + br_ref[...]


def _k6_call(c, h1, w2, b2_2d, maskcol, wr, br_2d):
    bk = 1280
    return pl.pallas_call(
        _k6_body,
        grid=(PN // bk,),
        in_specs=[
            pl.BlockSpec((G, bk), lambda i: (0, i)),
            pl.BlockSpec((bk, D), lambda i: (i, 0)),
            pl.BlockSpec((D, D), lambda i: (0, 0)),
            pl.BlockSpec((1, D), lambda i: (0, 0)),
            pl.BlockSpec((G, 1), lambda i: (0, 0)),
            pl.BlockSpec((D, 1), lambda i: (0, 0)),
            pl.BlockSpec((1, 1), lambda i: (0, 0)),
        ],
        out_specs=pl.BlockSpec((G, 1), lambda i: (0, 0)),
        out_shape=jax.ShapeDtypeStruct((G, 1), jnp.float32),
        scratch_shapes=[pltpu.VMEM((G, D), jnp.float32)],
    )(c, h1, w2, b2_2d, maskcol, wr, br_2d)


@jax.jit
def kernel(x, edge_index, batch, W1, b1, W2, b2, Wr, br):
    src = edge_index[0].astype(jnp.int32)
    dst = edge_index[1].astype(jnp.int32)
    bat = batch.astype(jnp.int32)
    e = src.shape[0]

    loop = jnp.arange(N, dtype=jnp.int32)
    n1 = -(-e // (NW * CHUNK))
    pe1 = n1 * NW * CHUNK
    src1 = _pad_edges(src, pe1, N).reshape(NW, n1, CHUNK)
    dst1 = _pad_edges(dst, pe1, N).reshape(NW, n1, CHUNK)
    srcx = jnp.concatenate([src, loop])
    dstx = jnp.concatenate([dst, loop])
    n2 = -(-srcx.shape[0] // (NW * CHUNK))
    pe2 = n2 * NW * CHUNK
    srcx = _pad_edges(srcx, pe2, N).reshape(NW, n2, CHUNK)
    dstx = _pad_edges(dstx, pe2, N).reshape(NW, n2, CHUNK)
    nb = -(-N // (NW * CHUNK))
    pb = nb * NW * CHUNK
    batp = _pad_edges(bat, pb, G).reshape(NW, nb, CHUNK)
    bat_node = _pad_edges(bat, PN, 0)

    zeros1d = jnp.zeros((GH * PN // NS,), jnp.float32)
    zeros2d = jnp.zeros((RPT, DH), jnp.float32)
    x_pad = jnp.concatenate([x, jnp.zeros((PN - N, D), x.dtype)])

    degp, cntp = _k1_call(dstx, batp, zeros1d, n2, nb)

    deg = degp[0] + degp[1]
    node_ok = (jnp.arange(PN) < N) & (deg > 0)
    dinv = jnp.where(node_ok, lax.rsqrt(jnp.maximum(deg, 1.0)), 0.0)
    dinv_col = dinv.reshape(PN, 1)
    cnt = cntp[0, :G] + cntp[1, :G]
    invcnt = jnp.where(cnt > 0, 1.0 / jnp.maximum(cnt, 1.0), 0.0)
    maskcol = (cnt > 0).astype(jnp.float32).reshape(G, 1)

    ycat, r_col = _k2_call(x_pad, W1, dinv_col, bat_node.reshape(PN, 1),
                           invcnt.reshape(G, 1))
    accp = _k4_call(src1, dst1, ycat.reshape(2 * PN, DH), zeros2d, n1)
    h1 = _k5_call(accp, ycat, dinv_col, b1.reshape(1, D))
    cp = _k3_call(srcx, dstx, dinv, r_col.reshape(PN), bat_node * PN,
                  zeros1d, n2)
    out = _k6_call(cp.reshape(G, PN), h1,
                   W2, b2.reshape(1, D), maskcol, Wr, br.reshape(1, 1))
    return out


# speedup vs baseline: 1.1509x; 1.1509x over previous
import functools

import jax
import jax.numpy as jnp
from jax import lax
from jax.experimental import pallas as pl
from jax.experimental.pallas import tpu as pltpu
from jax.experimental.pallas import tpu_sc as plsc

NC, NS, L = 2, 16, 16
NW = NC * NS
N = 10000
D = 128
DH = D // NC
G = 128
GH = G // NC
PN = 10240
CHUNK = 128
RPT = PN // NS

_MESH = plsc.VectorSubcoreMesh(
    core_axis_name="c", subcore_axis_name="s", num_cores=NC, num_subcores=NS)


def _pad_edges(idx, total, fill):
    return jnp.concatenate(
        [idx, jnp.full((total - idx.shape[0],), fill, dtype=idx.dtype)])


def _k1_body(dste_hbm, batp_hbm, zeros_hbm, degp_hbm, cntp_hbm,
             dst_v, bat_v, ones_v, deg_sh, cnt_sh, sem):
    cid = lax.axis_index("c")
    sid = lax.axis_index("s")
    wid = cid * NS + sid
    nde = dst_v.shape[0]
    nbt = bat_v.shape[0]

    pltpu.sync_copy(dste_hbm.at[wid], dst_v)
    pltpu.sync_copy(batp_hbm.at[wid], bat_v)
    for l in range(CHUNK // L):
        ones_v[pl.ds(l * L, L)] = jnp.ones((L,), jnp.float32)
    pltpu.sync_copy(zeros_hbm.at[pl.ds(0, PN // NS)],
                    deg_sh.at[pl.ds(sid * (PN // NS), PN // NS)])

    @pl.when(sid == 0)
    def _():
        pltpu.sync_copy(zeros_hbm.at[pl.ds(0, 256)], cnt_sh)

    plsc.subcore_barrier()

    def deg_step(j, _):
        pltpu.sync_copy(ones_v, deg_sh.at[dst_v.at[j]], add=True)
        return 0

    lax.fori_loop(0, nde, deg_step, 0)

    def cnt_step(j, _):
        pltpu.sync_copy(ones_v, cnt_sh.at[bat_v.at[j]], add=True)
        return 0

    lax.fori_loop(0, nbt, cnt_step, 0)
    plsc.subcore_barrier()

    pltpu.sync_copy(deg_sh.at[pl.ds(sid * (PN // NS), PN // NS)],
                    degp_hbm.at[cid, pl.ds(sid * (PN // NS), PN // NS)])

    @pl.when(sid == 0)
    def _():
        pltpu.sync_copy(cnt_sh, cntp_hbm.at[cid])


def _k1_call(dste, batp, zeros1d, n_dchunks, n_bchunks):
    return pl.kernel(
        _k1_body,
        out_type=[jax.ShapeDtypeStruct((NC, PN), jnp.float32),
                  jax.ShapeDtypeStruct((NC, 256), jnp.float32)],
        mesh=_MESH,
        scratch_types=[
            pltpu.VMEM((n_dchunks, CHUNK), jnp.int32),
            pltpu.VMEM((n_bchunks, CHUNK), jnp.int32),
            pltpu.VMEM((CHUNK,), jnp.float32),
            pltpu.VMEM_SHARED((PN,), jnp.float32),
            pltpu.VMEM_SHARED((256,), jnp.float32),
            pltpu.SemaphoreType.DMA,
        ],
    )(dste, batp, zeros1d)


def _k4_body(src_hbm, dst_hbm, ycat_hbm, zeros2d_hbm, accp_hbm,
             src_v, dst_v, rows_v, acc_sh, gsem, ssem):
    cid = lax.axis_index("c")
    sid = lax.axis_index("s")
    nch = src_v.shape[1]
    off = cid * PN

    pltpu.sync_copy(src_hbm.at[pl.ds(sid * 2, 2)], src_v)
    pltpu.sync_copy(dst_hbm.at[pl.ds(sid * 2, 2)], dst_v)
    pltpu.sync_copy(zeros2d_hbm, acc_sh.at[pl.ds(sid * RPT, RPT)])

    def bias_step(k, _):
        w = k // nch
        jj = k - w * nch
        for l in range(CHUNK // L):
            src_v[w, jj, pl.ds(l * L, L)] = (
                src_v[w, jj, pl.ds(l * L, L)] + off)
        return 0

    lax.fori_loop(0, 2 * nch, bias_step, 0)
    plsc.subcore_barrier()

    nt = 2 * nch

    def _gather(k, buf):
        w = k // nch
        jj = k - w * nch
        pltpu.async_copy(ycat_hbm.at[src_v.at[w, jj]], rows_v.at[buf], gsem)

    for b in range(3):
        _gather(b, b)

    def step(k, _):
        w = k // nch
        jj = k - w * nch
        cur = lax.rem(k, 4)
        pltpu.make_async_copy(ycat_hbm.at[src_v.at[w, jj]], rows_v.at[cur],
                              gsem).wait()
        pltpu.async_copy(rows_v.at[cur], acc_sh.at[dst_v.at[w, jj]], ssem,
                         add=True)

        @pl.when(k >= 1)
        def _():
            km = k - 1
            wm = km // nch
            jm = km - wm * nch
            pltpu.make_async_copy(rows_v.at[lax.rem(km, 4)],
                                  acc_sh.at[dst_v.at[wm, jm]], ssem).wait()

        @pl.when(k + 3 < nt)
        def _():
            k3 = k + 3
            w3 = k3 // nch
            j3 = k3 - w3 * nch
            pltpu.async_copy(ycat_hbm.at[src_v.at[w3, j3]],
                             rows_v.at[lax.rem(k3, 4)], gsem)

        return 0

    lax.fori_loop(0, nt, step, 0)
    pltpu.make_async_copy(rows_v.at[lax.rem(nt - 1, 4)],
                          acc_sh.at[dst_v.at[1, nch - 1]], ssem).wait()
    plsc.subcore_barrier()

    pltpu.sync_copy(acc_sh.at[pl.ds(sid * RPT, RPT)],
                    accp_hbm.at[cid, pl.ds(sid * RPT, RPT)])


def _k4_call(src1, dst1, ycat, zeros2d, n_chunks):
    return pl.kernel(
        _k4_body,
        out_type=jax.ShapeDtypeStruct((NC, PN, DH), jnp.float32),
        mesh=_MESH,
        scratch_types=[
            pltpu.VMEM((2, n_chunks, CHUNK), jnp.int32),
            pltpu.VMEM((2, n_chunks, CHUNK), jnp.int32),
            pltpu.VMEM((4, CHUNK, DH), jnp.float32),
            pltpu.VMEM_SHARED((PN, DH), jnp.float32),
            pltpu.SemaphoreType.DMA,
            pltpu.SemaphoreType.DMA,
        ],
        compiler_params=pltpu.CompilerParams(use_tc_tiling_on_sc=False),
    )(src1, dst1, ycat, zeros2d)


def _k3_body(src_hbm, dst_hbm, dinv_hbm, r_hbm, ibase_hbm, zeros_hbm, cp_hbm,
             src_v, dst_v, dvs_v, dvd_v, ib_v, val_v, idx_v, c_sh, gsem, ssem):
    cid = lax.axis_index("c")
    sid = lax.axis_index("s")
    wid = cid * NS + sid
    nt = src_v.shape[0]

    pltpu.sync_copy(src_hbm.at[wid], src_v)
    pltpu.sync_copy(dst_hbm.at[wid], dst_v)
    z = zeros_hbm.shape[0]
    csl = G * PN // NS
    for q in range(csl // z):
        pltpu.sync_copy(zeros_hbm,
                        c_sh.at[pl.ds(sid * csl + q * z, z)])
    plsc.subcore_barrier()

    def _gather(k, buf):
        pltpu.async_copy(dinv_hbm.at[src_v.at[k]], dvs_v.at[buf], gsem)
        pltpu.async_copy(r_hbm.at[dst_v.at[k]], dvd_v.at[buf], gsem)
        pltpu.async_copy(ibase_hbm.at[dst_v.at[k]], ib_v.at[buf], gsem)

    _gather(0, 0)
    _gather(1, 1)

    def step(j, _):
        cur = lax.rem(j, 2)
        pltpu.make_async_copy(dinv_hbm.at[src_v.at[j]], dvs_v.at[cur],
                              gsem).wait()
        pltpu.make_async_copy(r_hbm.at[dst_v.at[j]], dvd_v.at[cur],
                              gsem).wait()
        pltpu.make_async_copy(ibase_hbm.at[dst_v.at[j]], ib_v.at[cur],
                              gsem).wait()

        @pl.when(j >= 2)
        def _():
            pltpu.make_async_copy(val_v.at[cur], c_sh.at[idx_v.at[cur]],
                                  ssem).wait()

        for l in range(CHUNK // L):
            s = src_v[j, pl.ds(l * L, L)]
            ib = ib_v[cur, pl.ds(l * L, L)]
            v = dvs_v[cur, pl.ds(l * L, L)] * dvd_v[cur, pl.ds(l * L, L)]
            val_v[cur, pl.ds(l * L, L)] = v
            idx_v[cur, pl.ds(l * L, L)] = ib + s

        pltpu.async_copy(val_v.at[cur], c_sh.at[idx_v.at[cur]], ssem, add=True)

        @pl.when(j + 2 < nt)
        def _():
            _gather(j + 2, lax.rem(j + 2, 2))

        return 0

    lax.fori_loop(0, nt, step, 0)
    pltpu.make_async_copy(val_v.at[0], c_sh.at[idx_v.at[0]], ssem).wait()
    pltpu.make_async_copy(val_v.at[1], c_sh.at[idx_v.at[1]], ssem).wait()
    plsc.subcore_barrier()

    pltpu.sync_copy(c_sh.at[pl.ds(sid * csl, csl)],
                    cp_hbm.at[cid, pl.ds(sid * csl, csl)])


def _k3_call(srcx, dstx, dinv, r, ibase, zeros1d, n_chunks):
    return pl.kernel(
        _k3_body,
        out_type=jax.ShapeDtypeStruct((NC, G * PN), jnp.float32),
        mesh=_MESH,
        scratch_types=[
            pltpu.VMEM((n_chunks, CHUNK), jnp.int32),
            pltpu.VMEM((n_chunks, CHUNK), jnp.int32),
            pltpu.VMEM((2, CHUNK), jnp.float32),
            pltpu.VMEM((2, CHUNK), jnp.float32),
            pltpu.VMEM((2, CHUNK), jnp.int32),
            pltpu.VMEM((2, CHUNK), jnp.float32),
            pltpu.VMEM((2, CHUNK), jnp.int32),
            pltpu.VMEM_SHARED((G * PN,), jnp.float32),
            pltpu.SemaphoreType.DMA,
            pltpu.SemaphoreType.DMA,
        ],
        compiler_params=pltpu.CompilerParams(
            internal_scratch_in_bytes=3000000),
    )(srcx, dstx, dinv, r, ibase, zeros1d)


def _k2_body(x_ref, w1_ref, dinv_ref, bat_ref, icnt_ref, y_ref, r_ref):
    xw = jnp.dot(x_ref[...], w1_ref[...], preferred_element_type=jnp.float32)
    y = xw * dinv_ref[...]
    y_ref[0] = y[:, :DH]
    y_ref[1] = y[:, DH:]
    bm = bat_ref.shape[0]
    iota_g = lax.broadcasted_iota(jnp.int32, (bm, G), 1)
    oh = (bat_ref[...] == iota_g).astype(jnp.float32)
    icnt_node = jnp.dot(oh, icnt_ref[...], preferred_element_type=jnp.float32)
    r_ref[...] = dinv_ref[...] * icnt_node


def _k2_call(x_pad, w1, dinv_col, bat_col, icnt_col):
    bm = 512
    return pl.pallas_call(
        _k2_body,
        grid=(PN // bm,),
        in_specs=[
            pl.BlockSpec((bm, D), lambda i: (i, 0)),
            pl.BlockSpec((D, D), lambda i: (0, 0)),
            pl.BlockSpec((bm, 1), lambda i: (i, 0)),
            pl.BlockSpec((bm, 1), lambda i: (i, 0)),
            pl.BlockSpec((G, 1), lambda i: (0, 0)),
        ],
        out_specs=[pl.BlockSpec((2, bm, DH), lambda i: (0, i, 0)),
                   pl.BlockSpec((bm, 1), lambda i: (i, 0))],
        out_shape=[jax.ShapeDtypeStruct((2, PN, DH), jnp.float32),
                   jax.ShapeDtypeStruct((PN, 1), jnp.float32)],
    )(x_pad, w1, dinv_col, bat_col, icnt_col)


def _k6_body(c0_ref, c1_ref, accp_ref, y_ref, dinv_ref, b1_ref,
             w2_ref, b2_ref, mask_ref, wr_ref, br_ref, out_ref, qacc_ref):
    i = pl.program_id(0)

    @pl.when(i == 0)
    def _():
        qacc_ref[...] = jnp.zeros_like(qacc_ref)

    acc = jnp.concatenate([accp_ref[0], accp_ref[1]], axis=1)
    y = jnp.concatenate([y_ref[0], y_ref[1]], axis=1)
    h1 = jnp.maximum((acc + y) * dinv_ref[...] + b1_ref[...], 0.0)
    qacc_ref[...] += jnp.dot(c0_ref[...] + c1_ref[...], h1,
                             preferred_element_type=jnp.float32)

    @pl.when(i == pl.num_programs(0) - 1)
    def _():
        pooled = jnp.dot(qacc_ref[...], w2_ref[...],
                         preferred_element_type=jnp.float32) + b2_ref[...]
        pooled = pooled * mask_ref[...]
        h = jnp.maximum(pooled, 0.0)
        out_ref[...] = jnp.dot(h, wr_ref[...],
                               preferred_element_type=jnp.float32) + br_ref[...]


def _k6_call(c0, c1, accp, ycat, dinv_col, b1_2d, w2, b2_2d, maskcol, wr,
             br_2d):
    bk = 1280
    return pl.pallas_call(
        _k6_body,
        grid=(PN // bk,),
        in_specs=[
            pl.BlockSpec((G, bk), lambda i: (0, i)),
            pl.BlockSpec((G, bk), lambda i: (0, i)),
            pl.BlockSpec((2, bk, DH), lambda i: (0, i, 0)),
            pl.BlockSpec((2, bk, DH), lambda i: (0, i, 0)),
            pl.BlockSpec((bk, 1), lambda i: (i, 0)),
            pl.BlockSpec((1, D), lambda i: (0, 0)),
            pl.BlockSpec((D, D), lambda i: (0, 0)),
            pl.BlockSpec((1, D), lambda i: (0, 0)),
            pl.BlockSpec((G, 1), lambda i: (0, 0)),
            pl.BlockSpec((D, 1), lambda i: (0, 0)),
            pl.BlockSpec((1, 1), lambda i: (0, 0)),
        ],
        out_specs=pl.BlockSpec((G, 1), lambda i: (0, 0)),
        out_shape=jax.ShapeDtypeStruct((G, 1), jnp.float32),
        scratch_shapes=[pltpu.VMEM((G, D), jnp.float32)],
    )(c0, c1, accp, ycat, dinv_col, b1_2d, w2, b2_2d, maskcol, wr, br_2d)


@jax.jit
def kernel(x, edge_index, batch, W1, b1, W2, b2, Wr, br):
    src = edge_index[0].astype(jnp.int32)
    dst = edge_index[1].astype(jnp.int32)
    bat = batch.astype(jnp.int32)
    e = src.shape[0]

    loop = jnp.arange(N, dtype=jnp.int32)
    n1 = -(-e // (NW * CHUNK))
    pe1 = n1 * NW * CHUNK
    src1 = _pad_edges(src, pe1, N).reshape(NW, n1, CHUNK)
    dst1 = _pad_edges(dst, pe1, N).reshape(NW, n1, CHUNK)
    srcx = jnp.concatenate([src, loop])
    dstx = jnp.concatenate([dst, loop])
    n2 = -(-srcx.shape[0] // (NW * CHUNK))
    pe2 = n2 * NW * CHUNK
    srcx = _pad_edges(srcx, pe2, N).reshape(NW, n2, CHUNK)
    dstx = _pad_edges(dstx, pe2, N).reshape(NW, n2, CHUNK)
    nb = -(-N // (NW * CHUNK))
    pb = nb * NW * CHUNK
    batp = _pad_edges(bat, pb, G).reshape(NW, nb, CHUNK)
    bat_node = _pad_edges(bat, PN, 0)

    zeros1d = jnp.zeros((GH * PN // NS,), jnp.float32)
    zeros2d = jnp.zeros((RPT, DH), jnp.float32)
    x_pad = jnp.concatenate([x, jnp.zeros((PN - N, D), x.dtype)])

    degp, cntp = _k1_call(dstx, batp, zeros1d, n2, nb)

    deg = degp[0] + degp[1]
    node_ok = (jnp.arange(PN) < N) & (deg > 0)
    dinv = jnp.where(node_ok, lax.rsqrt(jnp.maximum(deg, 1.0)), 0.0)
    dinv_col = dinv.reshape(PN, 1)
    cnt = cntp[0, :G] + cntp[1, :G]
    invcnt = jnp.where(cnt > 0, 1.0 / jnp.maximum(cnt, 1.0), 0.0)
    maskcol = (cnt > 0).astype(jnp.float32).reshape(G, 1)

    ycat, r_col = _k2_call(x_pad, W1, dinv_col, bat_node.reshape(PN, 1),
                           invcnt.reshape(G, 1))
    accp = _k4_call(src1, dst1, ycat.reshape(2 * PN, DH), zeros2d, n1)
    cp = _k3_call(srcx, dstx, dinv, r_col.reshape(PN), bat_node * PN,
                  zeros1d, n2)
    out = _k6_call(cp[0].reshape(G, PN), cp[1].reshape(G, PN), accp, ycat,
                   dinv_col, b1.reshape(1, D), W2, b2.reshape(1, D), maskcol,
                   Wr, br.reshape(1, 1))
    return out
